# baseline (speedup 1.0000x reference)
import jax
import jax.numpy as jnp
from jax import lax
from jax.experimental import pallas as pl
from jax.experimental.pallas import tpu as pltpu
from jax.experimental.pallas import tpu_sc as plsc

N = 10000
E = 320000
D = 128
HD = D // 2
N_CLS = 7

NC = 2
NS = 16
CH = 128
RTOT = E // CH
RBASE = RTOT // NS
REXTRA = RTOT % NS
RMAX = RBASE + 1
N_PAD = 10240
STRIPE = N_PAD // NS

NBUF = 6
NGRP = RBASE // NBUF

_MESH = plsc.VectorSubcoreMesh(core_axis_name="c", subcore_axis_name="s")
_UNTILED = pltpu.CompilerParams(use_tc_tiling_on_sc=False)


def _row_split(sid):
    base = RBASE * sid + jnp.minimum(sid, REXTRA)
    cnt = jnp.where(sid < REXTRA, RBASE + 1, RBASE)
    return base, cnt


def _deg_body(edges_hbm, ones_hbm, zeros_hbm, out_hbm, dst_v, ones_v, acc_sh):
    cid = lax.axis_index("c")
    sid = lax.axis_index("s")
    base, cnt = _row_split(sid)
    half0 = (cnt + 1) // 2
    lo = base + cid * half0
    cnt_c = jnp.where(cid == 0, half0, cnt - half0)
    pltpu.sync_copy(edges_hbm.at[1, pl.ds(lo, RBASE // 2)],
                    dst_v.at[pl.ds(0, RBASE // 2)])
    pltpu.sync_copy(ones_hbm, ones_v)
    pltpu.sync_copy(zeros_hbm, acc_sh.at[pl.ds(sid * STRIPE, STRIPE)])

    @pl.when(cnt_c > RBASE // 2)
    def _():
        pltpu.sync_copy(edges_hbm.at[1, pl.ds(lo + RBASE // 2, 1)],
                        dst_v.at[pl.ds(RBASE // 2, 1)])

    plsc.subcore_barrier()

    def step(t, carry):
        pltpu.sync_copy(ones_v, acc_sh.at[dst_v.at[t]], add=True)
        return carry

    lax.fori_loop(0, cnt_c, step, 0)
    plsc.subcore_barrier()
    pltpu.sync_copy(acc_sh.at[pl.ds(sid * STRIPE, STRIPE)],
                    out_hbm.at[cid, pl.ds(sid * STRIPE, STRIPE)])


_deg_call = pl.kernel(
    _deg_body,
    out_type=jax.ShapeDtypeStruct((NC, N_PAD, 16), jnp.float32),
    mesh=_MESH,
    compiler_params=_UNTILED,
    scratch_types=[
        pltpu.VMEM((RBASE // 2 + 1, CH), jnp.int32),
        pltpu.VMEM((CH, 16), jnp.float32),
        pltpu.VMEM_SHARED((N_PAD, 16), jnp.float32),
    ],
)


def _agg_body(y0_hbm, y1_hbm, edges_hbm, zeros_hbm, out_hbm,
              src_v, dst_v, rows0, rows1, rows2, rows3,
              rows4, rows5,
              g0, g1, g2, g3, g4, g5, acc_sh):
    bufs = (rows0, rows1, rows2, rows3, rows4, rows5)
    gsems = (g0, g1, g2, g3, g4, g5)
    cid = lax.axis_index("c")
    sid = lax.axis_index("s")
    base, cnt = _row_split(sid)
    pltpu.sync_copy(edges_hbm.at[0, pl.ds(base, RBASE)],
                    src_v.at[pl.ds(0, RBASE)])
    pltpu.sync_copy(edges_hbm.at[1, pl.ds(base, RBASE)],
                    dst_v.at[pl.ds(0, RBASE)])
    pltpu.sync_copy(zeros_hbm, acc_sh.at[pl.ds(sid * STRIPE, STRIPE)])

    @pl.when(cnt > RBASE)
    def _():
        pltpu.sync_copy(edges_hbm.at[0, pl.ds(base + RBASE, 1)],
                        src_v.at[pl.ds(RBASE, 1)])
        pltpu.sync_copy(edges_hbm.at[1, pl.ds(base + RBASE, 1)],
                        dst_v.at[pl.ds(RBASE, 1)])

    plsc.subcore_barrier()

    def pipeline(tbl):
        def gstart(j, b):
            pltpu.async_copy(tbl.at[src_v.at[j]], bufs[b], gsems[b])

        def gwait(j, b):
            pltpu.make_async_copy(tbl.at[src_v.at[j]], bufs[b],
                                  gsems[b]).wait()

        def scat(j, b):
            pltpu.sync_copy(bufs[b], acc_sh.at[dst_v.at[j]], add=True)

        for b in range(NBUF):
            gstart(b, b)

        def group(i, carry):
            j0 = NBUF * i
            for b in range(NBUF):
                gwait(j0 + b, b)
                scat(j0 + b, b)

                @pl.when(j0 + NBUF + b < cnt)
                def _(b=b):
                    gstart(j0 + NBUF + b, b)

            return carry

        lax.fori_loop(0, NGRP, group, 0)
        for jj in range(NGRP * NBUF, RBASE + 1):
            @pl.when(jj < cnt)
            def _(jj=jj):
                gwait(jj, jj % NBUF)
                scat(jj, jj % NBUF)

    @pl.when(cid == 0)
    def _():
        pipeline(y0_hbm)

    @pl.when(cid == 1)
    def _():
        pipeline(y1_hbm)

    plsc.subcore_barrier()
    pltpu.sync_copy(acc_sh.at[pl.ds(sid * STRIPE, STRIPE)],
                    out_hbm.at[cid, pl.ds(sid * STRIPE, STRIPE)])


_agg_call = pl.kernel(
    _agg_body,
    out_type=jax.ShapeDtypeStruct((NC, N_PAD, HD), jnp.float32),
    mesh=_MESH,
    compiler_params=_UNTILED,
    scratch_types=[
        pltpu.VMEM((RMAX, CH), jnp.int32),
        pltpu.VMEM((RMAX, CH), jnp.int32),
        pltpu.VMEM((CH, HD), jnp.float32),
        pltpu.VMEM((CH, HD), jnp.float32),
        pltpu.VMEM((CH, HD), jnp.float32),
        pltpu.VMEM((CH, HD), jnp.float32),
        pltpu.VMEM((CH, HD), jnp.float32),
        pltpu.VMEM((CH, HD), jnp.float32),
        pltpu.SemaphoreType.DMA,
        pltpu.SemaphoreType.DMA,
        pltpu.SemaphoreType.DMA,
        pltpu.SemaphoreType.DMA,
        pltpu.SemaphoreType.DMA,
        pltpu.SemaphoreType.DMA,
        pltpu.VMEM_SHARED((N_PAD, HD), jnp.float32),
    ],
)


BR = 1000


def _r_block(degp_ref):
    deg = 1.0 + degp_ref[0, :, 0] + degp_ref[1, :, 0]
    return lax.rsqrt(deg)[:, None]


def _tc_mm_body(x_ref, w_ref, xw_ref):
    xw_ref[...] = jnp.dot(x_ref[...], w_ref[...],
                          preferred_element_type=jnp.float32)


def _tc_a_body(xw_ref, degp_ref, y0_ref, y1_ref):
    y = xw_ref[...] * _r_block(degp_ref)
    y0_ref[...] = y[:, :HD]
    y1_ref[...] = y[:, HD:]


def _tc_mid_body(degp_ref, q_ref, y0_ref, y1_ref, b_ref, w_ref,
                 z0_ref, z1_ref):
    r = _r_block(degp_ref)
    agg = jnp.concatenate([q_ref[0] + y0_ref[...], q_ref[1] + y1_ref[...]],
                          axis=1)
    h = jnp.maximum(agg * r + b_ref[...], 0.0)
    z = jnp.dot(h, w_ref[...], preferred_element_type=jnp.float32) * r
    z0_ref[...] = z[:, :HD]
    z1_ref[...] = z[:, HD:]


def _tc_out_body(degp_ref, q_ref, y0_ref, y1_ref, b_ref, wc_ref, bc_ref,
                 o_ref):
    r = _r_block(degp_ref)
    agg = jnp.concatenate([q_ref[0] + y0_ref[...], q_ref[1] + y1_ref[...]],
                          axis=1)
    h = jnp.maximum(agg * r + b_ref[...], 0.0)
    o_ref[...] = jnp.dot(h, wc_ref[...],
                         preferred_element_type=jnp.float32) + bc_ref[...]


_degp_spec = pl.BlockSpec((2, BR, 16), lambda i: (0, i, 0))
_part_spec = pl.BlockSpec((2, BR, HD), lambda i: (0, i, 0))
_row_spec = pl.BlockSpec((BR, D), lambda i: (i, 0))
_half_spec = pl.BlockSpec((BR, HD), lambda i: (i, 0))
_full_spec = pl.BlockSpec((D, D), lambda i: (0, 0))
_bias_spec = pl.BlockSpec((1, D), lambda i: (0, 0))

_half_out = jax.ShapeDtypeStruct((N, HD), jnp.float32)

_tc_mm = pl.pallas_call(
    _tc_mm_body,
    grid=(N // BR,),
    in_specs=[_row_spec, _full_spec],
    out_specs=_row_spec,
    out_shape=jax.ShapeDtypeStruct((N, D), jnp.float32),
)

_tc_a = pl.pallas_call(
    _tc_a_body,
    grid=(N // BR,),
    in_specs=[_row_spec, _degp_spec],
    out_specs=[_half_spec, _half_spec],
    out_shape=[_half_out, _half_out],
)

_tc_mid = pl.pallas_call(
    _tc_mid_body,
    grid=(N // BR,),
    in_specs=[_degp_spec, _part_spec, _half_spec, _half_spec, _bias_spec,
              _full_spec],
    out_specs=[_half_spec, _half_spec],
    out_shape=[_half_out, _half_out],
)

_tc_out = pl.pallas_call(
    _tc_out_body,
    grid=(N // BR,),
    in_specs=[_degp_spec, _part_spec, _half_spec, _half_spec, _bias_spec,
              pl.BlockSpec((D, 8), lambda i: (0, 0)),
              pl.BlockSpec((1, 8), lambda i: (0, 0))],
    out_specs=pl.BlockSpec((BR, 8), lambda i: (i, 0)),
    out_shape=jax.ShapeDtypeStruct((N, 8), jnp.float32),
)


def kernel(x, edge_index, W1, b1, W2, b2, Wc, bc):
    edges = edge_index.reshape(2, RTOT, CH)
    zeros_h = jnp.zeros((STRIPE, HD), jnp.float32)
    zeros_16 = jnp.zeros((STRIPE, 16), jnp.float32)
    ones_16 = jnp.ones((CH, 16), jnp.float32)
    b1r = b1.reshape(1, D)
    b2r = b2.reshape(1, D)
    wcp = jnp.pad(Wc, ((0, 0), (0, 8 - N_CLS)))
    bcp = jnp.pad(bc, (0, 8 - N_CLS)).reshape(1, 8)

    degp = _deg_call(edges, ones_16, zeros_16)
    xw = _tc_mm(x, W1)
    y0, y1 = _tc_a(xw, degp)
    p = _agg_call(y0, y1, edges, zeros_h)
    z0, z1 = _tc_mid(degp, p, y0, y1, b1r, W2)
    q = _agg_call(z0, z1, edges, zeros_h)
    out = _tc_out(degp, q, z0, z1, b2r, wcp, bcp)
    return out[:, :N_CLS]


# speedup vs baseline: 1.0159x; 1.0159x over previous
import jax
import jax.numpy as jnp
from jax import lax
from jax.experimental import pallas as pl
from jax.experimental.pallas import tpu as pltpu
from jax.experimental.pallas import tpu_sc as plsc

N = 10000
E = 320000
D = 128
HD = D // 2
N_CLS = 7

NC = 2
NS = 16
CH = 128
RTOT = E // CH
RBASE = RTOT // NS
REXTRA = RTOT % NS
RMAX = RBASE + 1
N_PAD = 10240
STRIPE = N_PAD // NS

NBUF = 4
NGRP = RBASE // NBUF

_MESH = plsc.VectorSubcoreMesh(core_axis_name="c", subcore_axis_name="s")
_UNTILED = pltpu.CompilerParams(use_tc_tiling_on_sc=False)


def _row_split(sid):
    base = RBASE * sid + jnp.minimum(sid, REXTRA)
    cnt = jnp.where(sid < REXTRA, RBASE + 1, RBASE)
    return base, cnt


def _deg_body(edges_hbm, ones_hbm, zeros_hbm, out_hbm, dst_v, ones_v, acc_sh):
    cid = lax.axis_index("c")
    sid = lax.axis_index("s")
    base, cnt = _row_split(sid)
    half0 = (cnt + 1) // 2
    lo = base + cid * half0
    cnt_c = jnp.where(cid == 0, half0, cnt - half0)
    pltpu.sync_copy(edges_hbm.at[1, pl.ds(lo, RBASE // 2)],
                    dst_v.at[pl.ds(0, RBASE // 2)])
    pltpu.sync_copy(ones_hbm, ones_v)
    pltpu.sync_copy(zeros_hbm, acc_sh.at[pl.ds(sid * STRIPE, STRIPE)])

    @pl.when(cnt_c > RBASE // 2)
    def _():
        pltpu.sync_copy(edges_hbm.at[1, pl.ds(lo + RBASE // 2, 1)],
                        dst_v.at[pl.ds(RBASE // 2, 1)])

    plsc.subcore_barrier()

    def step(t, carry):
        pltpu.sync_copy(ones_v, acc_sh.at[dst_v.at[t]], add=True)
        return carry

    lax.fori_loop(0, cnt_c, step, 0)
    plsc.subcore_barrier()
    pltpu.sync_copy(acc_sh.at[pl.ds(sid * STRIPE, STRIPE)],
                    out_hbm.at[cid, pl.ds(sid * STRIPE, STRIPE)])


_deg_call = pl.kernel(
    _deg_body,
    out_type=jax.ShapeDtypeStruct((NC, N_PAD, 16), jnp.float32),
    mesh=_MESH,
    compiler_params=_UNTILED,
    scratch_types=[
        pltpu.VMEM((RBASE // 2 + 1, CH), jnp.int32),
        pltpu.VMEM((CH, 16), jnp.float32),
        pltpu.VMEM_SHARED((N_PAD, 16), jnp.float32),
    ],
)


def _agg_body(y0_hbm, y1_hbm, edges_hbm, zeros_hbm, out_hbm,
              src_v, dst_v, rows0, rows1, rows2, rows3,
              g0, g1, g2, g3, acc_sh):
    bufs = (rows0, rows1, rows2, rows3)
    gsems = (g0, g1, g2, g3)
    cid = lax.axis_index("c")
    sid = lax.axis_index("s")
    base, cnt = _row_split(sid)
    pltpu.sync_copy(edges_hbm.at[0, pl.ds(base, RBASE)],
                    src_v.at[pl.ds(0, RBASE)])
    pltpu.sync_copy(edges_hbm.at[1, pl.ds(base, RBASE)],
                    dst_v.at[pl.ds(0, RBASE)])
    pltpu.sync_copy(zeros_hbm, acc_sh.at[pl.ds(sid * STRIPE, STRIPE)])

    @pl.when(cnt > RBASE)
    def _():
        pltpu.sync_copy(edges_hbm.at[0, pl.ds(base + RBASE, 1)],
                        src_v.at[pl.ds(RBASE, 1)])
        pltpu.sync_copy(edges_hbm.at[1, pl.ds(base + RBASE, 1)],
                        dst_v.at[pl.ds(RBASE, 1)])

    plsc.subcore_barrier()

    def pipeline(tbl):
        def gstart(j, b):
            pltpu.async_copy(tbl.at[src_v.at[j]], bufs[b], gsems[b])

        def gwait(j, b):
            pltpu.make_async_copy(tbl.at[src_v.at[j]], bufs[b],
                                  gsems[b]).wait()

        def scat(j, b):
            pltpu.sync_copy(bufs[b], acc_sh.at[dst_v.at[j]], add=True)

        for b in range(NBUF):
            gstart(b, b)

        def group(i, carry):
            j0 = NBUF * i
            for b in range(NBUF):
                gwait(j0 + b, b)
                scat(j0 + b, b)

                @pl.when(j0 + NBUF + b < cnt)
                def _(b=b):
                    gstart(j0 + NBUF + b, b)

            return carry

        lax.fori_loop(0, NGRP, group, 0)
        for jj in range(NGRP * NBUF, RBASE + 1):
            @pl.when(jj < cnt)
            def _(jj=jj):
                gwait(jj, jj % NBUF)
                scat(jj, jj % NBUF)

    @pl.when(cid == 0)
    def _():
        pipeline(y0_hbm)

    @pl.when(cid == 1)
    def _():
        pipeline(y1_hbm)

    plsc.subcore_barrier()
    pltpu.sync_copy(acc_sh.at[pl.ds(sid * STRIPE, STRIPE)],
                    out_hbm.at[cid, pl.ds(sid * STRIPE, STRIPE)])


_agg_call = pl.kernel(
    _agg_body,
    out_type=jax.ShapeDtypeStruct((NC, N_PAD, HD), jnp.float32),
    mesh=_MESH,
    compiler_params=_UNTILED,
    scratch_types=[
        pltpu.VMEM((RMAX, CH), jnp.int32),
        pltpu.VMEM((RMAX, CH), jnp.int32),
        pltpu.VMEM((CH, HD), jnp.float32),
        pltpu.VMEM((CH, HD), jnp.float32),
        pltpu.VMEM((CH, HD), jnp.float32),
        pltpu.VMEM((CH, HD), jnp.float32),
        pltpu.SemaphoreType.DMA,
        pltpu.SemaphoreType.DMA,
        pltpu.SemaphoreType.DMA,
        pltpu.SemaphoreType.DMA,
        pltpu.VMEM_SHARED((N_PAD, HD), jnp.float32),
    ],
)


BR = 2000


def _r_block(degp_ref):
    deg = 1.0 + degp_ref[0, :, 0] + degp_ref[1, :, 0]
    return lax.rsqrt(deg)[:, None]


def _tc_mm_body(x_ref, w_ref, xw_ref):
    xw_ref[...] = jnp.dot(x_ref[...], w_ref[...],
                          preferred_element_type=jnp.float32)


def _tc_a_body(xw_ref, degp_ref, y0_ref, y1_ref):
    y = xw_ref[...] * _r_block(degp_ref)
    y0_ref[...] = y[:, :HD]
    y1_ref[...] = y[:, HD:]


def _tc_mid_body(degp_ref, q_ref, y0_ref, y1_ref, b_ref, w_ref,
                 z0_ref, z1_ref):
    r = _r_block(degp_ref)
    agg = jnp.concatenate([q_ref[0] + y0_ref[...], q_ref[1] + y1_ref[...]],
                          axis=1)
    h = jnp.maximum(agg * r + b_ref[...], 0.0)
    z = jnp.dot(h, w_ref[...], preferred_element_type=jnp.float32) * r
    z0_ref[...] = z[:, :HD]
    z1_ref[...] = z[:, HD:]


def _tc_out_body(degp_ref, q_ref, y0_ref, y1_ref, b_ref, wc_ref, bc_ref,
                 o_ref):
    r = _r_block(degp_ref)
    agg = jnp.concatenate([q_ref[0] + y0_ref[...], q_ref[1] + y1_ref[...]],
                          axis=1)
    h = jnp.maximum(agg * r + b_ref[...], 0.0)
    o_ref[...] = jnp.dot(h, wc_ref[...],
                         preferred_element_type=jnp.float32) + bc_ref[...]


_degp_spec = pl.BlockSpec((2, BR, 16), lambda i: (0, i, 0))
_part_spec = pl.BlockSpec((2, BR, HD), lambda i: (0, i, 0))
_row_spec = pl.BlockSpec((BR, D), lambda i: (i, 0))
_half_spec = pl.BlockSpec((BR, HD), lambda i: (i, 0))
_full_spec = pl.BlockSpec((D, D), lambda i: (0, 0))
_bias_spec = pl.BlockSpec((1, D), lambda i: (0, 0))

_half_out = jax.ShapeDtypeStruct((N, HD), jnp.float32)

_tc_mm = pl.pallas_call(
    _tc_mm_body,
    grid=(N // BR,),
    in_specs=[_row_spec, _full_spec],
    out_specs=_row_spec,
    out_shape=jax.ShapeDtypeStruct((N, D), jnp.float32),
)

_tc_a = pl.pallas_call(
    _tc_a_body,
    grid=(N // BR,),
    in_specs=[_row_spec, _degp_spec],
    out_specs=[_half_spec, _half_spec],
    out_shape=[_half_out, _half_out],
)

_tc_mid = pl.pallas_call(
    _tc_mid_body,
    grid=(N // BR,),
    in_specs=[_degp_spec, _part_spec, _half_spec, _half_spec, _bias_spec,
              _full_spec],
    out_specs=[_half_spec, _half_spec],
    out_shape=[_half_out, _half_out],
)

_tc_out = pl.pallas_call(
    _tc_out_body,
    grid=(N // BR,),
    in_specs=[_degp_spec, _part_spec, _half_spec, _half_spec, _bias_spec,
              pl.BlockSpec((D, 8), lambda i: (0, 0)),
              pl.BlockSpec((1, 8), lambda i: (0, 0))],
    out_specs=pl.BlockSpec((BR, 8), lambda i: (i, 0)),
    out_shape=jax.ShapeDtypeStruct((N, 8), jnp.float32),
)


def kernel(x, edge_index, W1, b1, W2, b2, Wc, bc):
    edges = edge_index.reshape(2, RTOT, CH)
    zeros_h = jnp.zeros((STRIPE, HD), jnp.float32)
    zeros_16 = jnp.zeros((STRIPE, 16), jnp.float32)
    ones_16 = jnp.ones((CH, 16), jnp.float32)
    b1r = b1.reshape(1, D)
    b2r = b2.reshape(1, D)
    wcp = jnp.pad(Wc, ((0, 0), (0, 8 - N_CLS)))
    bcp = jnp.pad(bc, (0, 8 - N_CLS)).reshape(1, 8)

    degp = _deg_call(edges, ones_16, zeros_16)
    xw = _tc_mm(x, W1)
    y0, y1 = _tc_a(xw, degp)
    p = _agg_call(y0, y1, edges, zeros_h)
    z0, z1 = _tc_mid(degp, p, y0, y1, b1r, W2)
    q = _agg_call(z0, z1, edges, zeros_h)
    out = _tc_out(degp, q, z0, z1, b2r, wcp, bcp)
    return out[:, :N_CLS]


# speedup vs baseline: 1.0166x; 1.0007x over previous
import jax
import jax.numpy as jnp
from jax import lax
from jax.experimental import pallas as pl
from jax.experimental.pallas import tpu as pltpu
from jax.experimental.pallas import tpu_sc as plsc

N = 10000
E = 320000
D = 128
HD = D // 2
N_CLS = 7

NC = 2
NS = 16
CH = 128
RTOT = E // CH
RBASE = RTOT // NS
REXTRA = RTOT % NS
RMAX = RBASE + 1
N_PAD = 10240
STRIPE = N_PAD // NS

NBUF = 4
NGRP = RBASE // NBUF

_MESH = plsc.VectorSubcoreMesh(core_axis_name="c", subcore_axis_name="s")
_UNTILED = pltpu.CompilerParams(use_tc_tiling_on_sc=False)


def _row_split(sid):
    base = RBASE * sid + jnp.minimum(sid, REXTRA)
    cnt = jnp.where(sid < REXTRA, RBASE + 1, RBASE)
    return base, cnt


def _deg_body(edges_hbm, ones_hbm, zeros_hbm, out_hbm, dst_v, ones_v, acc_sh):
    cid = lax.axis_index("c")
    sid = lax.axis_index("s")
    base, cnt = _row_split(sid)
    half0 = (cnt + 1) // 2
    lo = base + cid * half0
    cnt_c = jnp.where(cid == 0, half0, cnt - half0)
    pltpu.sync_copy(edges_hbm.at[1, pl.ds(lo, RBASE // 2)],
                    dst_v.at[pl.ds(0, RBASE // 2)])
    pltpu.sync_copy(ones_hbm, ones_v)
    pltpu.sync_copy(zeros_hbm, acc_sh.at[pl.ds(sid * STRIPE, STRIPE)])

    @pl.when(cnt_c > RBASE // 2)
    def _():
        pltpu.sync_copy(edges_hbm.at[1, pl.ds(lo + RBASE // 2, 1)],
                        dst_v.at[pl.ds(RBASE // 2, 1)])

    plsc.subcore_barrier()

    def step(t, carry):
        pltpu.sync_copy(ones_v, acc_sh.at[dst_v.at[t]], add=True)
        return carry

    lax.fori_loop(0, cnt_c, step, 0)
    plsc.subcore_barrier()
    pltpu.sync_copy(acc_sh.at[pl.ds(sid * STRIPE, STRIPE)],
                    out_hbm.at[cid, pl.ds(sid * STRIPE, STRIPE)])


_deg_call = pl.kernel(
    _deg_body,
    out_type=jax.ShapeDtypeStruct((NC, N_PAD, 16), jnp.float32),
    mesh=_MESH,
    compiler_params=_UNTILED,
    scratch_types=[
        pltpu.VMEM((RBASE // 2 + 1, CH), jnp.int32),
        pltpu.VMEM((CH, 16), jnp.float32),
        pltpu.VMEM_SHARED((N_PAD, 16), jnp.float32),
    ],
)


def _agg_body(y0_hbm, y1_hbm, edges_hbm, zeros_hbm, out_hbm,
              src_v, dst_v, rows0, rows1, rows2, rows3,
              g0, g1, g2, g3, acc_sh):
    bufs = (rows0, rows1, rows2, rows3)
    gsems = (g0, g1, g2, g3)
    cid = lax.axis_index("c")
    sid = lax.axis_index("s")
    base, cnt = _row_split(sid)
    pltpu.sync_copy(edges_hbm.at[0, pl.ds(base, RBASE)],
                    src_v.at[pl.ds(0, RBASE)])
    pltpu.sync_copy(edges_hbm.at[1, pl.ds(base, RBASE)],
                    dst_v.at[pl.ds(0, RBASE)])
    pltpu.sync_copy(zeros_hbm, acc_sh.at[pl.ds(sid * STRIPE, STRIPE)])

    @pl.when(cnt > RBASE)
    def _():
        pltpu.sync_copy(edges_hbm.at[0, pl.ds(base + RBASE, 1)],
                        src_v.at[pl.ds(RBASE, 1)])
        pltpu.sync_copy(edges_hbm.at[1, pl.ds(base + RBASE, 1)],
                        dst_v.at[pl.ds(RBASE, 1)])

    plsc.subcore_barrier()

    def pipeline(tbl):
        def gstart(j, b):
            pltpu.async_copy(tbl.at[src_v.at[j]], bufs[b], gsems[b])

        def gwait(j, b):
            pltpu.make_async_copy(tbl.at[src_v.at[j]], bufs[b],
                                  gsems[b]).wait()

        def scat(j, b):
            pltpu.sync_copy(bufs[b], acc_sh.at[dst_v.at[j]], add=True)

        for b in range(NBUF):
            gstart(b, b)

        def group(i, carry):
            j0 = NBUF * i
            for b in range(NBUF):
                gwait(j0 + b, b)
                scat(j0 + b, b)

                @pl.when(j0 + NBUF + b < cnt)
                def _(b=b):
                    gstart(j0 + NBUF + b, b)

            return carry

        lax.fori_loop(0, NGRP, group, 0)
        for jj in range(NGRP * NBUF, RBASE + 1):
            @pl.when(jj < cnt)
            def _(jj=jj):
                gwait(jj, jj % NBUF)
                scat(jj, jj % NBUF)

    @pl.when(cid == 0)
    def _():
        pipeline(y0_hbm)

    @pl.when(cid == 1)
    def _():
        pipeline(y1_hbm)

    plsc.subcore_barrier()
    pltpu.sync_copy(acc_sh.at[pl.ds(sid * STRIPE, STRIPE)],
                    out_hbm.at[cid, pl.ds(sid * STRIPE, STRIPE)])


_agg_call = pl.kernel(
    _agg_body,
    out_type=jax.ShapeDtypeStruct((NC, N_PAD, HD), jnp.float32),
    mesh=_MESH,
    compiler_params=_UNTILED,
    scratch_types=[
        pltpu.VMEM((RMAX, CH), jnp.int32),
        pltpu.VMEM((RMAX, CH), jnp.int32),
        pltpu.VMEM((CH, HD), jnp.float32),
        pltpu.VMEM((CH, HD), jnp.float32),
        pltpu.VMEM((CH, HD), jnp.float32),
        pltpu.VMEM((CH, HD), jnp.float32),
        pltpu.SemaphoreType.DMA,
        pltpu.SemaphoreType.DMA,
        pltpu.SemaphoreType.DMA,
        pltpu.SemaphoreType.DMA,
        pltpu.VMEM_SHARED((N_PAD, HD), jnp.float32),
    ],
)


BR = 2000


def _r_block(degp_ref):
    deg = 1.0 + degp_ref[0, :, 0] + degp_ref[1, :, 0]
    return lax.rsqrt(deg)[:, None]


def _tc_mm_body(x_ref, w_ref, xw_ref):
    xw_ref[...] = jnp.dot(x_ref[...], w_ref[...],
                          preferred_element_type=jnp.float32)


def _tc_a_body(xw_ref, degp_ref, y0_ref, y1_ref):
    y = xw_ref[...] * _r_block(degp_ref)
    y0_ref[...] = y[:, :HD]
    y1_ref[...] = y[:, HD:]


def _tc_mid_body(degp_ref, q_ref, y0_ref, y1_ref, b_ref, w_ref,
                 z0_ref, z1_ref):
    r = _r_block(degp_ref)
    agg = jnp.concatenate([q_ref[0] + y0_ref[...], q_ref[1] + y1_ref[...]],
                          axis=1)
    h = jnp.maximum(agg * r + b_ref[...], 0.0)
    z = jnp.dot(h, w_ref[...], preferred_element_type=jnp.float32) * r
    z0_ref[...] = z[:, :HD]
    z1_ref[...] = z[:, HD:]


def _tc_out_body(degp_ref, q_ref, y0_ref, y1_ref, b_ref, wc_ref, bc_ref,
                 o_ref):
    r = _r_block(degp_ref)
    agg = jnp.concatenate([q_ref[0] + y0_ref[...], q_ref[1] + y1_ref[...]],
                          axis=1)
    h = jnp.maximum(agg * r + b_ref[...], 0.0)
    o_ref[...] = jnp.dot(h, wc_ref[...],
                         preferred_element_type=jnp.float32) + bc_ref[...]


_degp_spec = pl.BlockSpec((2, BR, 16), lambda i: (0, i, 0))
_part_spec = pl.BlockSpec((2, BR, HD), lambda i: (0, i, 0))
_row_spec = pl.BlockSpec((BR, D), lambda i: (i, 0))
_half_spec = pl.BlockSpec((BR, HD), lambda i: (i, 0))
_full_spec = pl.BlockSpec((D, D), lambda i: (0, 0))
_bias_spec = pl.BlockSpec((1, D), lambda i: (0, 0))

_half_out = jax.ShapeDtypeStruct((N, HD), jnp.float32)

_tc_mm = pl.pallas_call(
    _tc_mm_body,
    grid=(N // BR,),
    in_specs=[_row_spec, _full_spec],
    out_specs=_row_spec,
    out_shape=jax.ShapeDtypeStruct((N, D), jnp.float32),
)

_tc_a = pl.pallas_call(
    _tc_a_body,
    grid=(N // BR,),
    in_specs=[_row_spec, _degp_spec],
    out_specs=[_half_spec, _half_spec],
    out_shape=[_half_out, _half_out],
)

_tc_mid = pl.pallas_call(
    _tc_mid_body,
    grid=(N // BR,),
    in_specs=[_degp_spec, _part_spec, _half_spec, _half_spec, _bias_spec,
              _full_spec],
    out_specs=[_half_spec, _half_spec],
    out_shape=[_half_out, _half_out],
)

_tc_out = pl.pallas_call(
    _tc_out_body,
    grid=(N // BR,),
    in_specs=[_degp_spec, _part_spec, _half_spec, _half_spec, _bias_spec,
              pl.BlockSpec((D, N_CLS), lambda i: (0, 0)),
              pl.BlockSpec((1, N_CLS), lambda i: (0, 0))],
    out_specs=pl.BlockSpec((BR, N_CLS), lambda i: (i, 0)),
    out_shape=jax.ShapeDtypeStruct((N, N_CLS), jnp.float32),
)


def kernel(x, edge_index, W1, b1, W2, b2, Wc, bc):
    edges = edge_index.reshape(2, RTOT, CH)
    zeros_h = jnp.zeros((STRIPE, HD), jnp.float32)
    zeros_16 = jnp.zeros((STRIPE, 16), jnp.float32)
    ones_16 = jnp.ones((CH, 16), jnp.float32)
    b1r = b1.reshape(1, D)
    b2r = b2.reshape(1, D)
    bcr = bc.reshape(1, N_CLS)

    degp = _deg_call(edges, ones_16, zeros_16)
    xw = _tc_mm(x, W1)
    y0, y1 = _tc_a(xw, degp)
    p = _agg_call(y0, y1, edges, zeros_h)
    z0, z1 = _tc_mid(degp, p, y0, y1, b1r, W2)
    q = _agg_call(z0, z1, edges, zeros_h)
    return _tc_out(degp, q, z0, z1, b2r, Wc, bcr)
